# baseline (speedup 1.0000x reference)
import jax
import jax.numpy as jnp
from jax.experimental import pallas as pl

_SEQ = 32768
_K = 512
_R = 256
_W = 128


def _before(ka, ia, kb, ib):
    return (ka > kb) | ((ka == kb) & (ia < ib))


def _col_iota(shape, dim):
    return jax.lax.broadcasted_iota(jnp.int32, shape, dim)


def _ce_rows(kx, ix, d, ksz):
    n2 = _R // (2 * d)
    k4 = kx.reshape(n2, 2, d, _W)
    i4 = ix.reshape(n2, 2, d, _W)
    ka, kb = k4[:, 0], k4[:, 1]
    ia, ib = i4[:, 0], i4[:, 1]
    swb = _before(kb, ib, ka, ia)
    if ksz < _R:
        g = _col_iota((n2, 1, 1), 0)
        rev = ((g * (2 * d)) & ksz) != 0
    else:
        c = _col_iota((1, 1, _W), 2)
        rev = (c & (ksz // _R)) != 0
    do = jnp.logical_xor(swb, rev)
    nk = jnp.stack([jnp.where(do, kb, ka), jnp.where(do, ka, kb)], axis=1)
    ni = jnp.stack([jnp.where(do, ib, ia), jnp.where(do, ia, ib)], axis=1)
    return nk.reshape(_R, _W), ni.reshape(_R, _W)


def _roll_cols(x, m):
    return jnp.concatenate([x[:, m:], x[:, :m]], axis=1)


def _ce_cols(kx, ix, m, kszcol):
    col = _col_iota((_R, _W), 1)
    is_a = (col & m) == 0
    pk = jnp.where(is_a, _roll_cols(kx, m), _roll_cols(kx, _W - m))
    pi = jnp.where(is_a, _roll_cols(ix, m), _roll_cols(ix, _W - m))
    fk = jnp.where(is_a, kx, pk)
    fi = jnp.where(is_a, ix, pi)
    sk = jnp.where(is_a, pk, kx)
    si = jnp.where(is_a, pi, ix)
    swb = _before(sk, si, fk, fi)
    rev = (col & kszcol) != 0
    do = jnp.logical_xor(swb, rev)
    return jnp.where(do, pk, kx), jnp.where(do, pi, ix)


def _topk_body(scores_ref, iout_ref):
    x = scores_ref[...] + 0.0
    b = jax.lax.bitcast_convert_type(x, jnp.int32)
    kx = jnp.where(b >= 0, b, b ^ jnp.int32(0x7FFFFFFF))
    ix = _col_iota((_R, _W), 0) * _W + _col_iota((_R, _W), 1)

    ksz = 2
    while ksz <= _R:
        d = ksz // 2
        while d >= 1:
            kx, ix = _ce_rows(kx, ix, d, ksz)
            d //= 2
        ksz *= 2

    kx, ix = _ce_cols(kx, ix, 1, 2)
    d = _R // 2
    while d >= 1:
        kx, ix = _ce_rows(kx, ix, d, 512)
        d //= 2

    for t in range(6):
        lvl = 1024 << t
        kszcol = lvl // _R
        kx, ix = _ce_cols(kx, ix, lvl // 512, 0)
        kx, ix = _ce_cols(kx, ix, 1, kszcol)
        d = _R // 2
        while d >= 1:
            kx, ix = _ce_rows(kx, ix, d, lvl)
            d //= 2

    iout_ref[...] = ix[:, 0:2]


def kernel(scores):
    s2 = scores.reshape(_R, _W)
    idx = pl.pallas_call(
        _topk_body,
        out_shape=jax.ShapeDtypeStruct((_R, 2), jnp.int32),
    )(s2)
    return idx.T.reshape(_K)
